# baseline (speedup 1.0000x reference)
import functools

import jax
import jax.numpy as jnp
from jax import lax
from jax.experimental import pallas as pl
from jax.experimental.pallas import tpu as pltpu
from jax.experimental.pallas import tpu_sc as plsc

_SIZES = {"FILE": 10000, "CLASS": 50000, "FUNCTION": 100000}
_D = 128
_E = 40000
_RELS = [
    ("FILE", "OWNER", "CLASS", 0),
    ("FILE", "OWNER", "FUNCTION", 0),
    ("FILE", "CALL", "FUNCTION", 1),
    ("FILE", "IMPORT", "FILE", 1),
    ("FILE", "IMPORT", "CLASS", 1),
    ("FILE", "IMPORT", "FUNCTION", 1),
    ("CLASS", "OWNER", "CLASS", 2),
    ("CLASS", "OWNER", "FUNCTION", 2),
    ("FUNCTION", "OWNER", "CLASS", 2),
    ("FUNCTION", "OWNER", "FUNCTION", 2),
    ("CLASS", "CALL", "FUNCTION", 3),
    ("CLASS", "INHERITED", "CLASS", 3),
    ("FUNCTION", "CALL", "FUNCTION", 3),
]
_NREL = len(_RELS)

_NC, _NS = 2, 16
_NT = _NC * _NS
_G = 8
_L = 16

_PPAD = {"FILE": 320, "CLASS": 1568, "FUNCTION": 3136}
_PMAX = 3136

_GB = 2
_ECH = 2048
_EPAD = 40960
_NCHUNK = _EPAD // _ECH
_CAP = _EPAD + _ECH
_STG = 2 * _ECH
_BLK = 128

_BN = 2000


def _mm_kernel(x_ref, w_ref, o_ref):
    o_ref[...] = jnp.dot(x_ref[...], w_ref[...],
                         preferred_element_type=jnp.float32)


def _matmul(x, w):
    n = x.shape[0]
    return pl.pallas_call(
        _mm_kernel,
        grid=(n // _BN,),
        in_specs=[
            pl.BlockSpec((_BN, _D), lambda i: (i, 0)),
            pl.BlockSpec((_D, _D), lambda i: (0, 0)),
        ],
        out_specs=pl.BlockSpec((_BN, _D), lambda i: (i, 0)),
        out_shape=jax.ShapeDtypeStruct((n, _D), jnp.float32),
    )(x, w)


def _combine_kernel(inv_r, a0, a1, a2, a3, a4, a5, a6, a7, x_ref, w_ref,
                    b_ref, o_ref):
    agg = jnp.concatenate([a[0] for a in (a0, a1, a2, a3, a4, a5, a6, a7)],
                          axis=-1)
    dense = jnp.dot(x_ref[...], w_ref[...], preferred_element_type=jnp.float32)
    o_ref[...] = jnp.maximum(agg * inv_r + dense + b_ref[...], 0.0)


def _combine(acc8, x, w, b, num_rels):
    n = x.shape[0]
    aspec = [pl.BlockSpec((1, _BN, _L), lambda i, g=g: (g, i, 0))
             for g in range(_G)]
    return pl.pallas_call(
        functools.partial(_combine_kernel, 1.0 / num_rels),
        grid=(n // _BN,),
        in_specs=aspec + [
            pl.BlockSpec((_BN, _D), lambda i: (i, 0)),
            pl.BlockSpec((_D, _D), lambda i: (0, 0)),
            pl.BlockSpec((1, _D), lambda i: (0, 0)),
        ],
        out_specs=pl.BlockSpec((_BN, _D), lambda i: (i, 0)),
        out_shape=jax.ShapeDtypeStruct((n, _D), jnp.float32),
    )(*([acc8] * _G), x, w, b)


def _vgather(x, idx):
    dnums = lax.GatherDimensionNumbers(
        offset_dims=(), collapsed_slice_dims=(0,), start_index_map=(0,))
    return lax.gather(x, idx[:, None], dnums, (1,),
                      mode=lax.GatherScatterMode.PROMISE_IN_BOUNDS)


def _sc_mesh():
    return plsc.VectorSubcoreMesh(core_axis_name="c", subcore_axis_name="s",
                                  num_cores=_NC, num_subcores=_NS)


def _filter_body(*refs):
    ei_refs = refs[:2 * _NREL]
    z_hbm = refs[2 * _NREL]
    scr_dl, scr_src, scr_w, kblk_hbm = refs[2 * _NREL + 1:2 * _NREL + 5]
    csrc, cdst, fsrc, fdst, cnt, bdst, bw, kv = refs[2 * _NREL + 5:]

    wid = lax.axis_index("s") * _NC + lax.axis_index("c")
    lane = lax.iota(jnp.int32, _L)
    onehot0 = jnp.where(lane == 0, 1, 0)
    nblks = []

    for r, (_, _, d, _) in enumerate(_RELS):
        src_hbm, dst_hbm = ei_refs[2 * r], ei_refs[2 * r + 1]
        p_d = _PPAD[d]
        lo = wid * p_d
        hi = lo + p_d

        pltpu.sync_copy(z_hbm.at[pl.ds(0, p_d)], cnt.at[pl.ds(0, p_d)])

        def chunk_body(c, carry, src_hbm=src_hbm, dst_hbm=dst_hbm, lo=lo,
                       hi=hi, r=r):
            ptr0, nb = carry
            pltpu.sync_copy(src_hbm.at[pl.ds(c * _ECH, _ECH)], csrc)
            pltpu.sync_copy(dst_hbm.at[pl.ds(c * _ECH, _ECH)], cdst)

            def compact(i, p):
                dv = cdst[pl.ds(i * _L, _L)]
                sv = csrc[pl.ds(i * _L, _L)]
                m = (dv >= lo) & (dv < hi)
                mi = jnp.where(m, 1, 0)
                cum = mi
                for s in (1, 2, 4, 8):
                    g = _vgather(cum, jnp.maximum(lane - s, 0))
                    cum = cum + jnp.where(lane >= s, g, 0)
                pos = jnp.where(m, p + cum - mi, _STG)
                plsc.store_scatter(fdst, [pos], dv - lo)
                plsc.store_scatter(fsrc, [pos], sv)
                return p + plsc.all_reduce_population_count(m)[0]

            ptr1 = pl.loop(0, _ECH // _L, init_carry=ptr0)(compact)

            def count(j):
                dl = fdst[pl.ds(j, _L)][0]
                plsc.addupdate(cnt.at[pl.ds(dl, _L)], onehot0)

            pl.loop(ptr0, ptr1)(count)

            do_flush = ptr1 >= _ECH
            base = (wid * _NREL + r) * _CAP

            @pl.when(do_flush)
            def _():
                pltpu.sync_copy(fdst.at[pl.ds(0, _ECH)],
                                scr_dl.at[pl.ds(base + nb * _ECH, _ECH)])
                pltpu.sync_copy(fsrc.at[pl.ds(0, _ECH)],
                                scr_src.at[pl.ds(base + nb * _ECH, _ECH)])

                def shift(i):
                    fdst[pl.ds(i * _L, _L)] = fdst[pl.ds(_ECH + i * _L, _L)]
                    fsrc[pl.ds(i * _L, _L)] = fsrc[pl.ds(_ECH + i * _L, _L)]

                pl.loop(0, _ECH // _L)(shift)

            return (jnp.where(do_flush, ptr1 - _ECH, ptr1),
                    jnp.where(do_flush, nb + 1, nb))

        ptr, nb = pl.loop(0, _NCHUNK,
                          init_carry=(jnp.int32(0), jnp.int32(0)))(chunk_body)
        k = nb * _ECH + ptr

        def zpad(i, ptr=ptr):
            gi = i * _L + lane
            mz = gi >= ptr
            fdst[pl.ds(i * _L, _L)] = jnp.where(
                mz, 0, fdst[pl.ds(i * _L, _L)])
            fsrc[pl.ds(i * _L, _L)] = jnp.where(
                mz, 0, fsrc[pl.ds(i * _L, _L)])

        pl.loop(0, _ECH // _L)(zpad)

        rbase = (wid * _NREL + r) * _CAP

        @pl.when(ptr > 0)
        def _(nb=nb, rbase=rbase):
            pltpu.sync_copy(fdst.at[pl.ds(0, _ECH)],
                            scr_dl.at[pl.ds(rbase + nb * _ECH, _ECH)])
            pltpu.sync_copy(fsrc.at[pl.ds(0, _ECH)],
                            scr_src.at[pl.ds(rbase + nb * _ECH, _ECH)])

        nblk = (k + _BLK - 1) // _BLK
        nblks.append(nblk)

        def wpass(b, k=k, rbase=rbase):
            pltpu.sync_copy(scr_dl.at[pl.ds(rbase + b * _BLK, _BLK)], bdst)
            for i in range(_BLK // _L):
                dl16 = bdst[pl.ds(i * _L, _L)]
                c16 = plsc.load_gather(cnt, [dl16])
                w16 = 1.0 / jnp.maximum(c16.astype(jnp.float32), 1.0)
                valid = (b * _BLK + i * _L + lane) < k
                bw[pl.ds(i * _L, _L)] = jnp.where(valid, w16, 0.0)
            pltpu.sync_copy(bw, scr_w.at[pl.ds(rbase + b * _BLK, _BLK)])

        pl.loop(0, nblk)(wpass)

    kvv = jnp.zeros((_L,), jnp.int32)
    for r, nb_r in enumerate(nblks):
        kvv = kvv + jnp.where(lane == r, nb_r, 0)
    kv[pl.ds(0, _L)] = kvv
    pltpu.sync_copy(kv, kblk_hbm.at[pl.ds(wid * _L, _L)])


def _make_accum(y_order, rels_of):
    ny = len(y_order)
    mesh = _sc_mesh()
    out_type = [jax.ShapeDtypeStruct((_G * _NT * _PPAD[d] * _L,), jnp.float32)
                for d in ("FILE", "CLASS", "FUNCTION")]
    scratch = [
        pltpu.VMEM((_BLK,), jnp.int32),
        pltpu.VMEM((_BLK,), jnp.int32),
        pltpu.VMEM((_BLK,), jnp.float32),
        pltpu.VMEM((_BLK, _D), jnp.float32),
        pltpu.VMEM((_GB * _PMAX * _L,), jnp.float32),
        pltpu.VMEM((_L,), jnp.int32),
        pltpu.SemaphoreType.DMA,
    ]

    def body(*refs):
        y_refs = refs[:ny]
        scr_dl, scr_src, scr_w, kblk_hbm, z_hbm = refs[ny:ny + 5]
        outs = {"FILE": refs[ny + 5], "CLASS": refs[ny + 6],
                "FUNCTION": refs[ny + 7]}
        bdl, bsrc, bw, rows, acc, kv, sem = refs[ny + 8:]

        wid = lax.axis_index("s") * _NC + lax.axis_index("c")
        pltpu.sync_copy(kblk_hbm.at[pl.ds(wid * _L, _L)], kv)
        kvv = kv[pl.ds(0, _L)]

        for d in ("FILE", "CLASS", "FUNCTION"):
            p_d = _PPAD[d]
            out_hbm = outs[d]
            rel_list = rels_of[d]

            def g_body(pg, d=d, p_d=p_d, out_hbm=out_hbm, rel_list=rel_list):
                for q in range(_GB):
                    pltpu.sync_copy(
                        z_hbm.at[pl.ds(0, p_d * _L)],
                        acc.at[pl.ds(q * _PMAX * _L, p_d * _L)])
                for (r, yi) in rel_list:
                    y_hbm = y_refs[yi]
                    nblk = kvv[r]
                    rbase = (wid * _NREL + r) * _CAP

                    def blk_body(b, pg=pg, rbase=rbase, y_hbm=y_hbm):
                        pltpu.sync_copy(
                            scr_dl.at[pl.ds(rbase + b * _BLK, _BLK)], bdl)
                        pltpu.sync_copy(
                            scr_src.at[pl.ds(rbase + b * _BLK, _BLK)], bsrc)
                        pltpu.sync_copy(
                            scr_w.at[pl.ds(rbase + b * _BLK, _BLK)], bw)
                        pltpu.async_copy(y_hbm.at[bsrc], rows, sem).wait()

                        def sub16(i, pg=pg):
                            w16 = bw[pl.ds(i * _L, _L)]
                            dl16 = bdl[pl.ds(i * _L, _L)]
                            for j in range(_L):
                                dl_j = dl16[j]
                                w_j = w16[j]
                                for q in range(_GB):
                                    v = rows[i * _L + j,
                                             pl.ds((pg * _GB + q) * _L, _L)]
                                    plsc.addupdate(
                                        acc.at[pl.ds(
                                            (q * _PMAX + dl_j) * _L, _L)],
                                        v * w_j)

                        pl.loop(0, _BLK // _L)(sub16)

                    pl.loop(0, nblk)(blk_body)
                for q in range(_GB):
                    pltpu.sync_copy(
                        acc.at[pl.ds(q * _PMAX * _L, p_d * _L)],
                        out_hbm.at[pl.ds(
                            ((pg * _GB + q) * _NT + wid) * p_d * _L,
                            p_d * _L)])

            pl.loop(0, _G // _GB)(g_body)

    return pl.kernel(
        body, out_type=out_type, mesh=mesh, scratch_types=scratch,
        compiler_params=pltpu.CompilerParams(needs_layout_passes=False),
        name="sage_sc_accum")


def _make_filter():
    mesh = _sc_mesh()
    out_type = [
        jax.ShapeDtypeStruct((_NT * _NREL * _CAP,), jnp.int32),
        jax.ShapeDtypeStruct((_NT * _NREL * _CAP,), jnp.int32),
        jax.ShapeDtypeStruct((_NT * _NREL * _CAP,), jnp.float32),
        jax.ShapeDtypeStruct((_NT * _L,), jnp.int32),
    ]
    scratch = [
        pltpu.VMEM((_ECH,), jnp.int32),
        pltpu.VMEM((_ECH,), jnp.int32),
        pltpu.VMEM((_STG + _L,), jnp.int32),
        pltpu.VMEM((_STG + _L,), jnp.int32),
        pltpu.VMEM((_PMAX + _L,), jnp.int32),
        pltpu.VMEM((_BLK,), jnp.int32),
        pltpu.VMEM((_BLK,), jnp.float32),
        pltpu.VMEM((_L,), jnp.int32),
    ]
    return pl.kernel(
        _filter_body, out_type=out_type, mesh=mesh, scratch_types=scratch,
        compiler_params=pltpu.CompilerParams(needs_layout_passes=False),
        name="sage_sc_filter")


def kernel(x_FILE, x_CLASS, x_FUNCTION, ei_FILE_OWNER_CLASS, ei_FILE_OWNER_FUNCTION, ei_FILE_CALL_FUNCTION, ei_FILE_IMPORT_FILE, ei_FILE_IMPORT_CLASS, ei_FILE_IMPORT_FUNCTION, ei_CLASS_OWNER_CLASS, ei_CLASS_OWNER_FUNCTION, ei_FUNCTION_OWNER_CLASS, ei_FUNCTION_OWNER_FUNCTION, ei_CLASS_CALL_FUNCTION, ei_CLASS_INHERITED_CLASS, ei_FUNCTION_CALL_FUNCTION, Wl, bl, Wr):
    x = {"FILE": x_FILE, "CLASS": x_CLASS, "FUNCTION": x_FUNCTION}
    ei = {
        ("FILE", "OWNER", "CLASS"): ei_FILE_OWNER_CLASS,
        ("FILE", "OWNER", "FUNCTION"): ei_FILE_OWNER_FUNCTION,
        ("FILE", "CALL", "FUNCTION"): ei_FILE_CALL_FUNCTION,
        ("FILE", "IMPORT", "FILE"): ei_FILE_IMPORT_FILE,
        ("FILE", "IMPORT", "CLASS"): ei_FILE_IMPORT_CLASS,
        ("FILE", "IMPORT", "FUNCTION"): ei_FILE_IMPORT_FUNCTION,
        ("CLASS", "OWNER", "CLASS"): ei_CLASS_OWNER_CLASS,
        ("CLASS", "OWNER", "FUNCTION"): ei_CLASS_OWNER_FUNCTION,
        ("FUNCTION", "OWNER", "CLASS"): ei_FUNCTION_OWNER_CLASS,
        ("FUNCTION", "OWNER", "FUNCTION"): ei_FUNCTION_OWNER_FUNCTION,
        ("CLASS", "CALL", "FUNCTION"): ei_CLASS_CALL_FUNCTION,
        ("CLASS", "INHERITED", "CLASS"): ei_CLASS_INHERITED_CLASS,
        ("FUNCTION", "CALL", "FUNCTION"): ei_FUNCTION_CALL_FUNCTION,
    }

    Wl1, bl1, Wr1 = Wl[1], bl[1], Wr[1]

    pairs = sorted({(s, ci) for (s, _, _, ci) in _RELS})
    y2 = {(s, ci): _matmul(x[s], Wl1[ci]) for (s, ci) in pairs}
    y_order = pairs
    y_index = {p: i for i, p in enumerate(pairs)}

    srcs, dsts = [], []
    for (s, r, d, ci) in _RELS:
        e = ei[(s, r, d)]
        srcs.append(jnp.pad(e[0], (0, _EPAD - _E)))
        dsts.append(jnp.pad(e[1], (0, _EPAD - _E),
                            constant_values=jnp.int32(2 ** 30)))

    z_i32 = jnp.zeros((_PMAX,), jnp.int32)
    z_rows = jnp.zeros((_PMAX * _L,), jnp.float32)

    filt = _make_filter()
    ei_args = []
    for a, b in zip(srcs, dsts):
        ei_args.extend([a, b])
    scr_dl, scr_src, scr_w, kblk = filt(*ei_args, z_i32)

    rels_of = {"FILE": [], "CLASS": [], "FUNCTION": []}
    for r, (s, _, d, ci) in enumerate(_RELS):
        rels_of[d].append((r, y_index[(s, ci)]))

    accum = _make_accum(y_order, rels_of)
    acc8_F, acc8_C, acc8_FN = accum(
        *[y2[p] for p in y_order], scr_dl, scr_src, scr_w, kblk, z_rows)
    acc8 = {"FILE": acc8_F, "CLASS": acc8_C, "FUNCTION": acc8_FN}
    acc8 = {d: a.reshape(_G, _NT * _PPAD[d], _L) for d, a in acc8.items()}

    out = {}
    for nt in _SIZES:
        cis = [ci for (_, _, d, ci) in _RELS if d == nt]
        w_eff = sum(Wr1[ci] for ci in cis) / len(cis)
        b_eff = (sum(bl1[ci] for ci in cis) / len(cis)).reshape(1, _D)
        out[nt] = _combine(acc8[nt], x[nt], w_eff, b_eff, len(cis))
    return (out["FILE"], out["CLASS"], out["FUNCTION"])
